# Initial kernel scaffold; baseline (speedup 1.0000x reference)
import jax
import jax.numpy as jnp
from jax.experimental import pallas as pl


def kernel(edge_features, bond_type_table, stereo_table, W_binary, b_binary):
    raise NotImplementedError("write your pallas kernel here")


# speedup vs baseline: 1.6405x; 1.6405x over previous
import functools

import jax
import jax.numpy as jnp
from jax import lax
from jax.experimental import pallas as pl
from jax.experimental.pallas import tpu as pltpu
from jax.experimental.pallas import tpu_sc as plsc

NUM_EDGES = 320000
DIM = 128
L = 16

NC = 2
NS = 16
NW = NC * NS
EW = NUM_EDGES // NW
CHUNK = 400
NCHUNK = EW // CHUNK
NGROUP = CHUNK // L
NBT = 10
NST = 7
NROWS = NBT * NST
NCG = DIM // L


def _splat(val, dtype=jnp.int32):
    return jnp.full((L,), val, dtype)


@functools.partial(
    pl.kernel,
    out_type=jax.ShapeDtypeStruct((NUM_EDGES, DIM), jnp.float32),
    mesh=plsc.VectorSubcoreMesh(core_axis_name="c", subcore_axis_name="s"),
    compiler_params=pltpu.CompilerParams(needs_layout_passes=False),
    scratch_types=[
        pltpu.VMEM((CHUNK * 4,), jnp.float32),
        pltpu.VMEM((CHUNK, DIM), jnp.float32),
        pltpu.VMEM((NROWS * DIM,), jnp.float32),
        pltpu.VMEM((NBT, DIM), jnp.float32),
        pltpu.VMEM((NST, DIM), jnp.float32),
        pltpu.VMEM((2, DIM), jnp.float32),
        pltpu.VMEM((DIM,), jnp.float32),
    ],
)
def _sc_embed(feat_hbm, btab_hbm, stab_hbm, w_hbm, b_hbm, out_hbm,
              feat_v, out_v, tab_v, btab_v, stab_v, w_v, b_v):
    wid = lax.axis_index("s") * NC + lax.axis_index("c")

    pltpu.sync_copy(btab_hbm, btab_v)
    pltpu.sync_copy(stab_hbm, stab_v)
    pltpu.sync_copy(w_hbm, w_v)
    pltpu.sync_copy(b_hbm, b_v)

    iots = [lax.iota(jnp.int32, L) + cg * L for cg in range(NCG)]

    def build_row(i, carry):
        for cg in range(NCG):
            sl = pl.ds(cg * L, L)
            base = btab_v[i, sl] + b_v[sl]
            for j in range(NST):
                tab_v[pl.ds((i * NST + j) * DIM + cg * L, L)] = base + stab_v[j, sl]
        return carry
    lax.fori_loop(0, NBT, build_row, 0)

    def chunk_body(ch, carry):
        ebase = wid * EW + ch * CHUNK
        pltpu.sync_copy(feat_hbm.at[pl.ds(ebase * 4, CHUNK * 4)], feat_v)

        def group_body(g, gcarry):
            gb = g * L
            w0s = [w_v[0, pl.ds(cg * L, L)] for cg in range(NCG)]
            w1s = [w_v[1, pl.ds(cg * L, L)] for cg in range(NCG)]
            for e in range(L):
                base4 = (gb + e) * 4
                f0 = plsc.load_gather(feat_v, [_splat(base4) + 0])
                f1 = plsc.load_gather(feat_v, [_splat(base4) + 1])
                f2 = plsc.load_gather(feat_v, [_splat(base4) + 2])
                f3 = plsc.load_gather(feat_v, [_splat(base4) + 3])
                bt = jnp.clip((f0 * 2.0).astype(jnp.int32), 0, NBT - 1)
                st = jnp.clip(f3.astype(jnp.int32), 0, NST - 1)
                rbase = (bt * NST + st) * DIM
                for cg in range(NCG):
                    tg = plsc.load_gather(tab_v, [rbase + iots[cg]])
                    out_v[gb + e, pl.ds(cg * L, L)] = tg + f1 * w0s[cg] + f2 * w1s[cg]
            return gcarry
        lax.fori_loop(0, NGROUP, group_body, 0)

        pltpu.sync_copy(out_v, out_hbm.at[pl.ds(ebase, CHUNK)])
        return carry
    lax.fori_loop(0, NCHUNK, chunk_body, 0)


def kernel(edge_features, bond_type_table, stereo_table, W_binary, b_binary):
    return _sc_embed(edge_features.reshape(-1), bond_type_table, stereo_table,
                     W_binary.T, b_binary)


# speedup vs baseline: 1.9080x; 1.1630x over previous
import functools

import jax
import jax.numpy as jnp
from jax import lax
from jax.experimental import pallas as pl
from jax.experimental.pallas import tpu as pltpu
from jax.experimental.pallas import tpu_sc as plsc

NUM_EDGES = 320000
DIM = 128
L = 16

NC = 2
NS = 16
NW = NC * NS
EW = NUM_EDGES // NW
CHUNK = 400
NCHUNK = EW // CHUNK
NGROUP = CHUNK // L
NBT = 10
NST = 7
NROWS = NBT * NST
NCG = DIM // L


def _splat(val, dtype=jnp.int32):
    return jnp.full((L,), val, dtype)


@functools.partial(
    pl.kernel,
    out_type=jax.ShapeDtypeStruct((NUM_EDGES, DIM), jnp.float32),
    mesh=plsc.VectorSubcoreMesh(core_axis_name="c", subcore_axis_name="s"),
    compiler_params=pltpu.CompilerParams(needs_layout_passes=False),
    scratch_types=[
        pltpu.VMEM((CHUNK * 4,), jnp.float32),
        pltpu.VMEM((CHUNK, DIM), jnp.float32),
        pltpu.VMEM((NROWS * DIM,), jnp.float32),
        pltpu.VMEM((NBT, DIM), jnp.float32),
        pltpu.VMEM((NST, DIM), jnp.float32),
        pltpu.VMEM((2, DIM), jnp.float32),
        pltpu.VMEM((DIM,), jnp.float32),
    ],
)
def _sc_embed(feat_hbm, btab_hbm, stab_hbm, w_hbm, b_hbm, out_hbm,
              feat_v, out_v, tab_v, btab_v, stab_v, w_v, b_v):
    wid = lax.axis_index("s") * NC + lax.axis_index("c")

    pltpu.sync_copy(btab_hbm, btab_v)
    pltpu.sync_copy(stab_hbm, stab_v)
    pltpu.sync_copy(w_hbm, w_v)
    pltpu.sync_copy(b_hbm, b_v)

    iots = [lax.iota(jnp.int32, L) + cg * L for cg in range(NCG)]
    iota4 = lax.iota(jnp.int32, L) * 4
    w0s = [w_v[0, pl.ds(cg * L, L)] for cg in range(NCG)]
    w1s = [w_v[1, pl.ds(cg * L, L)] for cg in range(NCG)]

    def build_row(i, carry):
        for cg in range(NCG):
            sl = pl.ds(cg * L, L)
            base = btab_v[i, sl] + b_v[sl]
            for j in range(NST):
                tab_v[pl.ds((i * NST + j) * DIM + cg * L, L)] = base + stab_v[j, sl]
        return carry
    lax.fori_loop(0, NBT, build_row, 0)

    def chunk_body(ch, carry):
        ebase = wid * EW + ch * CHUNK
        pltpu.sync_copy(feat_hbm.at[pl.ds(ebase * 4, CHUNK * 4)], feat_v)

        def group_body(g, gcarry):
            gb = g * L
            fbase = _splat(gb * 4) + iota4
            f0v = plsc.load_gather(feat_v, [fbase])
            f1v = plsc.load_gather(feat_v, [fbase + 1])
            f2v = plsc.load_gather(feat_v, [fbase + 2])
            f3v = plsc.load_gather(feat_v, [fbase + 3])
            btv = jnp.clip((f0v * 2.0).astype(jnp.int32), 0, NBT - 1)
            stv = jnp.clip(f3v.astype(jnp.int32), 0, NST - 1)
            rbasev = (btv * NST + stv) * DIM
            for e in range(L):
                rbase = jnp.full((L,), rbasev[e])
                f1 = jnp.full((L,), f1v[e])
                f2 = jnp.full((L,), f2v[e])
                for cg in range(NCG):
                    tg = plsc.load_gather(tab_v, [rbase + iots[cg]])
                    out_v[gb + e, pl.ds(cg * L, L)] = tg + f1 * w0s[cg] + f2 * w1s[cg]
            return gcarry
        lax.fori_loop(0, NGROUP, group_body, 0)

        pltpu.sync_copy(out_v, out_hbm.at[pl.ds(ebase, CHUNK)])
        return carry
    lax.fori_loop(0, NCHUNK, chunk_body, 0)


def kernel(edge_features, bond_type_table, stereo_table, W_binary, b_binary):
    return _sc_embed(edge_features.reshape(-1), bond_type_table, stereo_table,
                     W_binary.T, b_binary)


# speedup vs baseline: 3.5816x; 1.8772x over previous
import functools

import jax
import jax.numpy as jnp
from jax import lax
from jax.experimental import pallas as pl
from jax.experimental.pallas import tpu as pltpu
from jax.experimental.pallas import tpu_sc as plsc

NUM_EDGES = 320000
DIM = 128
L = 16

NC = 2
NS = 16
NW = NC * NS
EW = NUM_EDGES // NW
CHUNK = 400
NCHUNK = EW // CHUNK
NGROUP = CHUNK // L
NBT = 10
NST = 7
NROWS = NBT * NST
NCG = DIM // L


def _splat(val, dtype=jnp.int32):
    return jnp.full((L,), val, dtype)


@functools.partial(
    pl.kernel,
    out_type=jax.ShapeDtypeStruct((NUM_EDGES, DIM), jnp.float32),
    mesh=plsc.VectorSubcoreMesh(core_axis_name="c", subcore_axis_name="s"),
    compiler_params=pltpu.CompilerParams(needs_layout_passes=False),
    scratch_types=[
        pltpu.VMEM((CHUNK * 4,), jnp.float32),
        pltpu.VMEM((CHUNK, DIM), jnp.float32),
        pltpu.VMEM((NROWS * DIM,), jnp.float32),
        pltpu.VMEM((NBT, DIM), jnp.float32),
        pltpu.VMEM((NST, DIM), jnp.float32),
        pltpu.VMEM((2, DIM), jnp.float32),
        pltpu.VMEM((DIM,), jnp.float32),
    ],
)
def _sc_embed(feat_hbm, btab_hbm, stab_hbm, w_hbm, b_hbm, out_hbm,
              feat_v, out_v, tab_v, btab_v, stab_v, w_v, b_v):
    wid = lax.axis_index("s") * NC + lax.axis_index("c")

    pltpu.sync_copy(btab_hbm, btab_v)
    pltpu.sync_copy(stab_hbm, stab_v)
    pltpu.sync_copy(w_hbm, w_v)
    pltpu.sync_copy(b_hbm, b_v)

    iota = lax.iota(jnp.int32, L)
    iota4 = iota * 4
    tab_cg = [tab_v.at[pl.ds(cg * L, NROWS * DIM - (NCG - 1) * L)]
              for cg in range(NCG)]
    w0s = [w_v[0, pl.ds(cg * L, L)] for cg in range(NCG)]
    w1s = [w_v[1, pl.ds(cg * L, L)] for cg in range(NCG)]

    def build_row(i, carry):
        for cg in range(NCG):
            sl = pl.ds(cg * L, L)
            base = btab_v[i, sl] + b_v[sl]
            for j in range(NST):
                tab_v[pl.ds((i * NST + j) * DIM + cg * L, L)] = base + stab_v[j, sl]
        return carry
    lax.fori_loop(0, NBT, build_row, 0)

    def chunk_body(ch, carry):
        ebase = wid * EW + ch * CHUNK
        pltpu.sync_copy(feat_hbm.at[pl.ds(ebase * 4, CHUNK * 4)], feat_v)

        def group_body(g, gcarry):
            gb = g * L
            fbase = _splat(gb * 4) + iota4
            f0v = plsc.load_gather(feat_v, [fbase])
            f1v = plsc.load_gather(feat_v, [fbase + 1])
            f2v = plsc.load_gather(feat_v, [fbase + 2])
            f3v = plsc.load_gather(feat_v, [fbase + 3])
            btv = jnp.clip((f0v * 2.0).astype(jnp.int32), 0, NBT - 1)
            stv = jnp.clip(f3v.astype(jnp.int32), 0, NST - 1)
            rbasev = (btv * NST + stv) * DIM
            for e in range(L):
                rbase = jnp.full((L,), rbasev[e]) + iota
                f1 = jnp.full((L,), f1v[e])
                f2 = jnp.full((L,), f2v[e])
                tgs = [plsc.load_gather(tab_cg[cg], [rbase]) for cg in range(NCG)]
                for cg in range(NCG):
                    out_v[gb + e, pl.ds(cg * L, L)] = (
                        (tgs[cg] + f1 * w0s[cg]) + f2 * w1s[cg])
            return gcarry
        lax.fori_loop(0, NGROUP, group_body, 0)

        pltpu.sync_copy(out_v, out_hbm.at[pl.ds(ebase, CHUNK)])
        return carry
    lax.fori_loop(0, NCHUNK, chunk_body, 0)


def kernel(edge_features, bond_type_table, stereo_table, W_binary, b_binary):
    return _sc_embed(edge_features.reshape(-1), bond_type_table, stereo_table,
                     W_binary.T, b_binary)


# speedup vs baseline: 4.2020x; 1.1732x over previous
import functools

import jax
import jax.numpy as jnp
from jax import lax
from jax.experimental import pallas as pl
from jax.experimental.pallas import tpu as pltpu
from jax.experimental.pallas import tpu_sc as plsc

NUM_EDGES = 320000
DIM = 128
L = 16

NC = 2
NS = 16
NW = NC * NS
EW = NUM_EDGES // NW
CHUNK = 400
NCHUNK = EW // CHUNK
NGROUP = CHUNK // L
NBT = 10
NST = 7
NROWS = NBT * NST
NCG = DIM // L


def _splat(val, dtype=jnp.int32):
    return jnp.full((L,), val, dtype)


@functools.partial(
    pl.kernel,
    out_type=jax.ShapeDtypeStruct((NUM_EDGES, DIM), jnp.float32),
    mesh=plsc.VectorSubcoreMesh(core_axis_name="c", subcore_axis_name="s"),
    compiler_params=pltpu.CompilerParams(needs_layout_passes=False),
    scratch_types=[
        pltpu.VMEM((CHUNK, 4), jnp.float32),
        pltpu.VMEM((CHUNK, DIM), jnp.float32),
        pltpu.VMEM((NROWS * DIM,), jnp.float32),
        pltpu.VMEM((NBT, DIM), jnp.float32),
        pltpu.VMEM((NST, DIM), jnp.float32),
        pltpu.VMEM((2, DIM), jnp.float32),
        pltpu.VMEM((DIM,), jnp.float32),
    ],
)
def _sc_embed(feat_hbm, btab_hbm, stab_hbm, w_hbm, b_hbm, out_hbm,
              feat_v, out_v, tab_v, btab_v, stab_v, w_v, b_v):
    wid = lax.axis_index("s") * NC + lax.axis_index("c")

    pltpu.sync_copy(btab_hbm, btab_v)
    pltpu.sync_copy(stab_hbm, stab_v)
    pltpu.sync_copy(w_hbm, w_v)
    pltpu.sync_copy(b_hbm, b_v)

    iota = lax.iota(jnp.int32, L)
    iota4 = iota * 4
    tab_cg = [tab_v.at[pl.ds(cg * L, NROWS * DIM - (NCG - 1) * L)]
              for cg in range(NCG)]
    w0s = [w_v[0, pl.ds(cg * L, L)] for cg in range(NCG)]
    w1s = [w_v[1, pl.ds(cg * L, L)] for cg in range(NCG)]

    def build_row(i, carry):
        for cg in range(NCG):
            sl = pl.ds(cg * L, L)
            base = btab_v[i, sl] + b_v[sl]
            for j in range(NST):
                tab_v[pl.ds((i * NST + j) * DIM + cg * L, L)] = base + stab_v[j, sl]
        return carry
    lax.fori_loop(0, NBT, build_row, 0)

    def chunk_body(ch, carry):
        ebase = wid * EW + ch * CHUNK
        pltpu.sync_copy(feat_hbm.at[pl.ds(ebase, CHUNK)], feat_v)

        def group_body(g, gcarry):
            gb = g * L
            rows = _splat(gb) + iota
            f0v = plsc.load_gather(feat_v, [rows, _splat(0)])
            f1v = plsc.load_gather(feat_v, [rows, _splat(1)])
            f2v = plsc.load_gather(feat_v, [rows, _splat(2)])
            f3v = plsc.load_gather(feat_v, [rows, _splat(3)])
            btv = jnp.clip((f0v * 2.0).astype(jnp.int32), 0, NBT - 1)
            stv = jnp.clip(f3v.astype(jnp.int32), 0, NST - 1)
            rbasev = (btv * NST + stv) * DIM
            for e in range(L):
                rbase = jnp.full((L,), rbasev[e]) + iota
                f1 = jnp.full((L,), f1v[e])
                f2 = jnp.full((L,), f2v[e])
                tgs = [plsc.load_gather(tab_cg[cg], [rbase]) for cg in range(NCG)]
                for cg in range(NCG):
                    out_v[gb + e, pl.ds(cg * L, L)] = (
                        (tgs[cg] + f1 * w0s[cg]) + f2 * w1s[cg])
            return gcarry
        lax.fori_loop(0, NGROUP, group_body, 0)

        pltpu.sync_copy(out_v, out_hbm.at[pl.ds(ebase, CHUNK)])
        return carry
    lax.fori_loop(0, NCHUNK, chunk_body, 0)


def kernel(edge_features, bond_type_table, stereo_table, W_binary, b_binary):
    return _sc_embed(edge_features, bond_type_table, stereo_table,
                     W_binary.T, b_binary)


# speedup vs baseline: 6.7246x; 1.6003x over previous
import functools

import jax
import jax.numpy as jnp
from jax import lax
from jax.experimental import pallas as pl
from jax.experimental.pallas import tpu as pltpu
from jax.experimental.pallas import tpu_sc as plsc

NUM_EDGES = 320000
DIM = 128
L = 16
BLK = 128

NC = 2
NS = 16
NW = NC * NS
NBLK = NUM_EDGES // BLK
CB = 2
CHUNK = CB * BLK
NCHUNKS = NBLK // CB
NGROUP = CHUNK // L
NBT = 10
NST = 7
NROWS = NBT * NST
NCG = DIM // L


@functools.partial(
    pl.kernel,
    out_type=jax.ShapeDtypeStruct((NUM_EDGES, DIM), jnp.float32),
    mesh=plsc.VectorSubcoreMesh(core_axis_name="c", subcore_axis_name="s"),
    compiler_params=pltpu.CompilerParams(needs_layout_passes=False),
    scratch_types=[
        pltpu.VMEM((CB, 4, BLK), jnp.float32),
        pltpu.VMEM((CHUNK, DIM), jnp.float32),
        pltpu.VMEM((NROWS * DIM,), jnp.float32),
        pltpu.VMEM((NBT, DIM), jnp.float32),
        pltpu.VMEM((NST, DIM), jnp.float32),
        pltpu.VMEM((2, DIM), jnp.float32),
        pltpu.VMEM((DIM,), jnp.float32),
    ],
)
def _sc_embed(feat_hbm, btab_hbm, stab_hbm, w_hbm, b_hbm, out_hbm,
              feat_v, out_v, tab_v, btab_v, stab_v, w_v, b_v):
    wid = lax.axis_index("s") * NC + lax.axis_index("c")

    pltpu.sync_copy(btab_hbm, btab_v)
    pltpu.sync_copy(stab_hbm, stab_v)
    pltpu.sync_copy(w_hbm, w_v)
    pltpu.sync_copy(b_hbm, b_v)

    iota = lax.iota(jnp.int32, L)
    tab_cg = [tab_v.at[pl.ds(cg * L, NROWS * DIM - (NCG - 1) * L)]
              for cg in range(NCG)]

    w0s = [w_v[0, pl.ds(cg * L, L)] for cg in range(NCG)]
    w1s = [w_v[1, pl.ds(cg * L, L)] for cg in range(NCG)]

    def build_row(i, carry):
        for cg in range(NCG):
            sl = pl.ds(cg * L, L)
            base = btab_v[i, sl] + b_v[sl]
            for j in range(NST):
                tab_v[pl.ds((i * NST + j) * DIM + cg * L, L)] = base + stab_v[j, sl]
        return carry
    lax.fori_loop(0, NBT, build_row, 0)

    n_mine = NCHUNKS // NW + jnp.where(wid < NCHUNKS % NW, 1, 0)

    def chunk_body(k, carry):
        ci = wid + k * NW
        ebase = ci * CHUNK
        pltpu.sync_copy(feat_hbm.at[pl.ds(ci * CB, CB)], feat_v)

        def group_body(g, gcarry):
            gb = g * L
            bi = g // (BLK // L)
            lo = (g % (BLK // L)) * L
            f0v = feat_v[bi, 0, pl.ds(lo, L)]
            f1v = feat_v[bi, 1, pl.ds(lo, L)]
            f2v = feat_v[bi, 2, pl.ds(lo, L)]
            f3v = feat_v[bi, 3, pl.ds(lo, L)]
            btv = jnp.clip((f0v * 2.0).astype(jnp.int32), 0, NBT - 1)
            stv = jnp.clip(f3v.astype(jnp.int32), 0, NST - 1)
            rbasev = (btv * NST + stv) * DIM
            for e in range(L):
                rbase = jnp.full((L,), rbasev[e]) + iota
                f1 = jnp.full((L,), f1v[e])
                f2 = jnp.full((L,), f2v[e])
                tgs = [plsc.load_gather(tab_cg[cg], [rbase]) for cg in range(NCG)]
                for cg in range(NCG):
                    out_v[gb + e, pl.ds(cg * L, L)] = (
                        (tgs[cg] + f1 * w0s[cg]) + f2 * w1s[cg])
            return gcarry
        lax.fori_loop(0, NGROUP, group_body, 0)

        pltpu.sync_copy(out_v, out_hbm.at[pl.ds(ebase, CHUNK)])
        return carry
    lax.fori_loop(0, n_mine, chunk_body, 0)


def kernel(edge_features, bond_type_table, stereo_table, W_binary, b_binary):
    feat3 = edge_features.reshape(NBLK, BLK, 4).transpose(0, 2, 1)
    return _sc_embed(feat3, bond_type_table, stereo_table,
                     W_binary.T, b_binary)


# speedup vs baseline: 10.5012x; 1.5616x over previous
import functools

import jax
import jax.numpy as jnp
from jax import lax
from jax.experimental import pallas as pl
from jax.experimental.pallas import tpu as pltpu
from jax.experimental.pallas import tpu_sc as plsc

NUM_EDGES = 320000
DIM = 128
L = 16
BLK = 128

NC = 2
NS = 16
NW = NC * NS
NBLK = NUM_EDGES // BLK
CB = 2
CHUNK = CB * BLK
NCHUNKS = NBLK // CB
NGROUP = CHUNK // L
NBT = 10
NST = 7
NROWS = NBT * NST
NCG = DIM // L


@functools.partial(
    pl.kernel,
    out_type=jax.ShapeDtypeStruct((NUM_EDGES, DIM), jnp.float32),
    mesh=plsc.VectorSubcoreMesh(core_axis_name="c", subcore_axis_name="s"),
    compiler_params=pltpu.CompilerParams(needs_layout_passes=False),
    scratch_types=[
        pltpu.VMEM((2, CB, 4, BLK), jnp.float32),
        pltpu.VMEM((2, CHUNK, DIM), jnp.float32),
        pltpu.VMEM((NROWS * DIM,), jnp.float32),
        pltpu.VMEM((NBT, DIM), jnp.float32),
        pltpu.VMEM((NST, DIM), jnp.float32),
        pltpu.VMEM((2, DIM), jnp.float32),
        pltpu.VMEM((DIM,), jnp.float32),
        pltpu.SemaphoreType.DMA,
        pltpu.SemaphoreType.DMA,
        pltpu.SemaphoreType.DMA,
        pltpu.SemaphoreType.DMA,
    ],
)
def _sc_embed(feat_hbm, btab_hbm, stab_hbm, w_hbm, b_hbm, out_hbm,
              feat_v, out_v, tab_v, btab_v, stab_v, w_v, b_v,
              sem_f0, sem_f1, sem_o0, sem_o1):
    wid = lax.axis_index("s") * NC + lax.axis_index("c")

    pltpu.sync_copy(btab_hbm, btab_v)
    pltpu.sync_copy(stab_hbm, stab_v)
    pltpu.sync_copy(w_hbm, w_v)
    pltpu.sync_copy(b_hbm, b_v)

    iota = lax.iota(jnp.int32, L)
    tab_cg = [tab_v.at[pl.ds(cg * L, NROWS * DIM - (NCG - 1) * L)]
              for cg in range(NCG)]

    w0s = [w_v[0, pl.ds(cg * L, L)] for cg in range(NCG)]
    w1s = [w_v[1, pl.ds(cg * L, L)] for cg in range(NCG)]

    def build_row(i, carry):
        for cg in range(NCG):
            sl = pl.ds(cg * L, L)
            base = btab_v[i, sl] + b_v[sl]
            for j in range(NST):
                tab_v[pl.ds((i * NST + j) * DIM + cg * L, L)] = base + stab_v[j, sl]
        return carry
    lax.fori_loop(0, NBT, build_row, 0)

    n_mine = NCHUNKS // NW + jnp.where(wid < NCHUNKS % NW, 1, 0)
    sems_f = [sem_f0, sem_f1]
    sems_o = [sem_o0, sem_o1]

    def feat_load(k, b):
        ci = wid + k * NW
        pltpu.async_copy(feat_hbm.at[pl.ds(ci * CB, CB)], feat_v.at[b],
                         sems_f[b])

    def compute_chunk(k, b):
        def group_body(g, gcarry):
            gb = g * L
            bi = g // (BLK // L)
            lo = (g % (BLK // L)) * L
            f0v = feat_v[b, bi, 0, pl.ds(lo, L)]
            f1v = feat_v[b, bi, 1, pl.ds(lo, L)]
            f2v = feat_v[b, bi, 2, pl.ds(lo, L)]
            f3v = feat_v[b, bi, 3, pl.ds(lo, L)]
            btv = jnp.clip((f0v * 2.0).astype(jnp.int32), 0, NBT - 1)
            stv = jnp.clip(f3v.astype(jnp.int32), 0, NST - 1)
            rbasev = (btv * NST + stv) * DIM
            for e in range(L):
                rbase = jnp.full((L,), rbasev[e]) + iota
                f1 = jnp.full((L,), f1v[e])
                f2 = jnp.full((L,), f2v[e])
                tgs = [plsc.load_gather(tab_cg[cg], [rbase]) for cg in range(NCG)]
                for cg in range(NCG):
                    out_v[b, gb + e, pl.ds(cg * L, L)] = (
                        (tgs[cg] + f1 * w0s[cg]) + f2 * w1s[cg])
            return gcarry
        lax.fori_loop(0, NGROUP, group_body, 0)

    def wait_feat(b):
        pltpu.make_async_copy(feat_hbm.at[pl.ds(0, CB)], feat_v.at[b],
                              sems_f[b]).wait()

    def wait_out(b):
        pltpu.make_async_copy(out_v.at[b], out_hbm.at[pl.ds(0, CHUNK)],
                              sems_o[b]).wait()

    feat_load(0, 0)
    feat_load(1, 1)

    def pair_body(k2, carry):
        for b in range(2):
            k = k2 * 2 + b

            @pl.when(k < n_mine)
            def _():
                wait_feat(b)

                @pl.when(k >= 2)
                def _():
                    wait_out(b)

                compute_chunk(k, b)
                ci = wid + k * NW
                pltpu.async_copy(out_v.at[b], out_hbm.at[pl.ds(ci * CHUNK, CHUNK)],
                                 sems_o[b])

                @pl.when(k + 2 < n_mine)
                def _():
                    feat_load(k + 2, b)
        return carry
    lax.fori_loop(0, (NCHUNKS // NW + 2) // 2, pair_body, 0)

    wait_out(0)
    wait_out(1)


def kernel(edge_features, bond_type_table, stereo_table, W_binary, b_binary):
    feat3 = edge_features.reshape(NBLK, BLK, 4).transpose(0, 2, 1)
    return _sc_embed(feat3, bond_type_table, stereo_table,
                     W_binary.T, b_binary)
